# baseline (speedup 1.0000x reference)
import functools
import jax
import jax.numpy as jnp
from jax.experimental import pallas as pl

N_BLOCK = 1000


def _embed_kernel(batch_ref, attrs_ref, feats_ref, fieldx_ref, ew_ref, out_ref):
    B = batch_ref.shape[1]
    G = fieldx_ref.shape[0]
    attrs = attrs_ref[0]
    mx = jnp.max(attrs, axis=1, keepdims=True)
    col_ids = jax.lax.broadcasted_iota(jnp.int32, attrs.shape, 1)
    idx = jnp.min(jnp.where(attrs == mx, col_ids, 127), axis=1, keepdims=True)
    s = jnp.zeros((B, 1), jnp.float32)
    for e in range(attrs.shape[1]):
        s = jnp.where(idx == e, ew_ref[0, e], s)
    b = batch_ref[0]
    g_ids = jax.lax.broadcasted_iota(jnp.int32, (B, G), 1)
    oh = jnp.where(g_ids == b, 1.0, 0.0).astype(jnp.bfloat16)
    mult = jax.lax.dot_general(
        oh, fieldx_ref[...], (((1,), (0,)), ((), ())),
        preferred_element_type=jnp.float32)
    out_ref[...] = feats_ref[...] + s * mult


@jax.jit
def kernel(batch, node_feats_1, node_attrs, data_external_field,
           element_weights, channel_weights):
    N, C, K = node_feats_1.shape
    G = data_external_field.shape[0]
    E = node_attrs.shape[1]
    feats = node_feats_1.reshape(N, C * K)
    nb = N // N_BLOCK
    batch_r = batch.astype(jnp.int32).reshape(nb, N_BLOCK, 1)
    attrs_r = node_attrs.reshape(nb, N_BLOCK, E)
    fieldx = (channel_weights[None, :, None]
              * data_external_field[:, None, :]).reshape(G, C * K)
    fieldx_bf16 = fieldx.astype(jnp.bfloat16)
    ew_pad = jnp.zeros((1, 128), jnp.float32).at[0, :E].set(element_weights)

    out = pl.pallas_call(
        _embed_kernel,
        grid=(nb,),
        in_specs=[
            pl.BlockSpec((1, N_BLOCK, 1), lambda i: (i, 0, 0)),
            pl.BlockSpec((1, N_BLOCK, E), lambda i: (i, 0, 0)),
            pl.BlockSpec((N_BLOCK, C * K), lambda i: (i, 0)),
            pl.BlockSpec((G, C * K), lambda i: (0, 0)),
            pl.BlockSpec((1, 128), lambda i: (0, 0)),
        ],
        out_specs=pl.BlockSpec((N_BLOCK, C * K), lambda i: (i, 0)),
        out_shape=jax.ShapeDtypeStruct((N, C * K), jnp.float32),
    )(batch_r, attrs_r, feats, fieldx_bf16, ew_pad)
    return out.reshape(N, C, K)


# speedup vs baseline: 4.2492x; 4.2492x over previous
import jax
import jax.numpy as jnp
from jax.experimental import pallas as pl

N_BLOCK = 1000


def _embed_kernel(batch_ref, attrs_ref, feats_ref, fieldx_ref, ew_ref, out_ref):
    B = batch_ref.shape[1]
    G = fieldx_ref.shape[0]
    C = feats_ref.shape[2]
    K = feats_ref.shape[0]
    attrs = attrs_ref[0]
    mx = jnp.max(attrs, axis=1, keepdims=True)
    col_ids = jax.lax.broadcasted_iota(jnp.int32, attrs.shape, 1)
    idx = jnp.min(jnp.where(attrs == mx, col_ids, 127), axis=1, keepdims=True)
    s = jnp.zeros((B, 1), jnp.float32)
    for e in range(attrs.shape[1]):
        s = jnp.where(idx == e, ew_ref[0, e], s)
    b = batch_ref[0]
    g_ids = jax.lax.broadcasted_iota(jnp.int32, (B, G), 1)
    oh = jnp.where(g_ids == b, 1.0, 0.0).astype(jnp.bfloat16)
    mult = jax.lax.dot_general(
        oh, fieldx_ref[...], (((1,), (0,)), ((), ())),
        preferred_element_type=jnp.float32)
    for k in range(K):
        out_ref[k] = feats_ref[k] + s * mult[:, k * C:(k + 1) * C]


@jax.jit
def kernel(batch, node_feats_1, node_attrs, data_external_field,
           element_weights, channel_weights):
    N, C, K = node_feats_1.shape
    G = data_external_field.shape[0]
    E = node_attrs.shape[1]
    feats_t = jnp.transpose(node_feats_1, (2, 0, 1))
    nb = N // N_BLOCK
    batch_r = batch.astype(jnp.int32).reshape(nb, N_BLOCK, 1)
    attrs_r = node_attrs.reshape(nb, N_BLOCK, E)
    fieldx = (data_external_field[:, :, None]
              * channel_weights[None, None, :]).reshape(G, K * C)
    fieldx_bf16 = fieldx.astype(jnp.bfloat16)
    ew_pad = jnp.zeros((1, 128), jnp.float32).at[0, :E].set(element_weights)

    out3 = pl.pallas_call(
        _embed_kernel,
        grid=(nb,),
        in_specs=[
            pl.BlockSpec((1, N_BLOCK, 1), lambda i: (i, 0, 0)),
            pl.BlockSpec((1, N_BLOCK, E), lambda i: (i, 0, 0)),
            pl.BlockSpec((K, N_BLOCK, C), lambda i: (0, i, 0)),
            pl.BlockSpec((G, K * C), lambda i: (0, 0)),
            pl.BlockSpec((1, 128), lambda i: (0, 0)),
        ],
        out_specs=pl.BlockSpec((K, N_BLOCK, C), lambda i: (0, i, 0)),
        out_shape=jax.ShapeDtypeStruct((K, N, C), jnp.float32),
    )(batch_r, attrs_r, feats_t, fieldx_bf16, ew_pad)
    return jnp.transpose(out3, (1, 2, 0))


# speedup vs baseline: 5.1694x; 1.2166x over previous
import jax
import jax.numpy as jnp
from jax.experimental import pallas as pl

N_BLOCK = 4000


def _embed_kernel(batch_ref, attrs_ref, feats_ref, fieldx_ref, ew_ref, out_ref):
    B = batch_ref.shape[1]
    G = fieldx_ref.shape[0]
    C = feats_ref.shape[2]
    K = feats_ref.shape[0]
    attrs = attrs_ref[0]
    mx = jnp.max(attrs, axis=1, keepdims=True)
    col_ids = jax.lax.broadcasted_iota(jnp.int32, attrs.shape, 1)
    idx = jnp.min(jnp.where(attrs == mx, col_ids, 127), axis=1, keepdims=True)
    s = jnp.zeros((B, 1), jnp.float32)
    for e in range(attrs.shape[1]):
        s = jnp.where(idx == e, ew_ref[0, e], s)
    b = batch_ref[0]
    g_ids = jax.lax.broadcasted_iota(jnp.int32, (B, G), 1)
    oh = jnp.where(g_ids == b, 1.0, 0.0).astype(jnp.bfloat16)
    mult = jax.lax.dot_general(
        oh, fieldx_ref[...], (((1,), (0,)), ((), ())),
        preferred_element_type=jnp.float32)
    for k in range(K):
        out_ref[k] = feats_ref[k] + s * mult[:, k * C:(k + 1) * C]


@jax.jit
def kernel(batch, node_feats_1, node_attrs, data_external_field,
           element_weights, channel_weights):
    N, C, K = node_feats_1.shape
    G = data_external_field.shape[0]
    E = node_attrs.shape[1]
    feats_t = jnp.transpose(node_feats_1, (2, 0, 1))
    nb = N // N_BLOCK
    batch_r = batch.astype(jnp.int32).reshape(nb, N_BLOCK, 1)
    attrs_r = node_attrs.reshape(nb, N_BLOCK, E)
    fieldx = (data_external_field[:, :, None]
              * channel_weights[None, None, :]).reshape(G, K * C)
    fieldx_bf16 = fieldx.astype(jnp.bfloat16)
    ew_pad = jnp.zeros((1, 128), jnp.float32).at[0, :E].set(element_weights)

    out3 = pl.pallas_call(
        _embed_kernel,
        grid=(nb,),
        in_specs=[
            pl.BlockSpec((1, N_BLOCK, 1), lambda i: (i, 0, 0)),
            pl.BlockSpec((1, N_BLOCK, E), lambda i: (i, 0, 0)),
            pl.BlockSpec((K, N_BLOCK, C), lambda i: (0, i, 0)),
            pl.BlockSpec((G, K * C), lambda i: (0, 0)),
            pl.BlockSpec((1, 128), lambda i: (0, 0)),
        ],
        out_specs=pl.BlockSpec((K, N_BLOCK, C), lambda i: (0, i, 0)),
        out_shape=jax.ShapeDtypeStruct((K, N, C), jnp.float32),
    )(batch_r, attrs_r, feats_t, fieldx_bf16, ew_pad)
    return jnp.transpose(out3, (1, 2, 0))


# speedup vs baseline: 5.2396x; 1.0136x over previous
import jax
import jax.numpy as jnp
from jax.experimental import pallas as pl

N_BLOCK = 5000


def _embed_kernel(batch_ref, attrs_ref, feats_ref, fieldx_ref, ew_ref, out_ref):
    B = batch_ref.shape[1]
    G = fieldx_ref.shape[0]
    C = feats_ref.shape[2]
    K = feats_ref.shape[0]
    attrs = attrs_ref[0]
    mx = jnp.max(attrs, axis=1, keepdims=True)
    col_ids = jax.lax.broadcasted_iota(jnp.int32, attrs.shape, 1)
    idx = jnp.min(jnp.where(attrs == mx, col_ids, 127), axis=1, keepdims=True)
    s = jnp.zeros((B, 1), jnp.float32)
    for e in range(attrs.shape[1]):
        s = jnp.where(idx == e, ew_ref[0, e], s)
    b = batch_ref[0]
    g_ids = jax.lax.broadcasted_iota(jnp.int32, (B, G), 1)
    oh = jnp.where(g_ids == b, 1.0, 0.0).astype(jnp.bfloat16)
    mult = jax.lax.dot_general(
        oh, fieldx_ref[...], (((1,), (0,)), ((), ())),
        preferred_element_type=jnp.float32)
    for k in range(K):
        out_ref[k] = feats_ref[k] + s * mult[:, k * C:(k + 1) * C]


@jax.jit
def kernel(batch, node_feats_1, node_attrs, data_external_field,
           element_weights, channel_weights):
    N, C, K = node_feats_1.shape
    G = data_external_field.shape[0]
    E = node_attrs.shape[1]
    feats_t = jnp.transpose(node_feats_1, (2, 0, 1))
    nb = N // N_BLOCK
    batch_r = batch.astype(jnp.int32).reshape(nb, N_BLOCK, 1)
    attrs_r = node_attrs.reshape(nb, N_BLOCK, E)
    fieldx = (data_external_field[:, :, None]
              * channel_weights[None, None, :]).reshape(G, K * C)
    fieldx_bf16 = fieldx.astype(jnp.bfloat16)
    ew_pad = jnp.zeros((1, 128), jnp.float32).at[0, :E].set(element_weights)

    out3 = pl.pallas_call(
        _embed_kernel,
        grid=(nb,),
        in_specs=[
            pl.BlockSpec((1, N_BLOCK, 1), lambda i: (i, 0, 0)),
            pl.BlockSpec((1, N_BLOCK, E), lambda i: (i, 0, 0)),
            pl.BlockSpec((K, N_BLOCK, C), lambda i: (0, i, 0)),
            pl.BlockSpec((G, K * C), lambda i: (0, 0)),
            pl.BlockSpec((1, 128), lambda i: (0, 0)),
        ],
        out_specs=pl.BlockSpec((K, N_BLOCK, C), lambda i: (0, i, 0)),
        out_shape=jax.ShapeDtypeStruct((K, N, C), jnp.float32),
    )(batch_r, attrs_r, feats_t, fieldx_bf16, ew_pad)
    return jnp.transpose(out3, (1, 2, 0))


# speedup vs baseline: 10.9809x; 2.0957x over previous
import jax
import jax.numpy as jnp
from jax.experimental import pallas as pl

N_BLOCK = 3968


def _embed_kernel(batch_ref, attrs_ref, feats_ref, fieldx_ref, ew_ref, out_ref):
    B = batch_ref.shape[1]
    G = fieldx_ref.shape[0]
    C = feats_ref.shape[2]
    K = feats_ref.shape[0]
    E = attrs_ref.shape[0]
    a = attrs_ref[...]
    mx = jnp.max(a, axis=0, keepdims=True)
    eq = a == mx
    s_row = jnp.zeros((1, B), jnp.float32)
    taken = jnp.zeros((1, B), jnp.bool_)
    for e in range(E):
        eq_e = eq[e:e + 1, :]
        s_row = jnp.where(eq_e & ~taken, ew_ref[0, e], s_row)
        taken = taken | eq_e
    b_row = batch_ref[...]
    g_ids = jax.lax.broadcasted_iota(jnp.int32, (G, B), 0)
    ohs = jnp.where(g_ids == b_row, s_row, 0.0).astype(jnp.bfloat16)
    mult = jax.lax.dot_general(
        ohs, fieldx_ref[...], (((0,), (0,)), ((), ())),
        preferred_element_type=jnp.float32)
    for k in range(K):
        out_ref[k] = feats_ref[k] + mult[:, k * C:(k + 1) * C]


@jax.jit
def kernel(batch, node_feats_1, node_attrs, data_external_field,
           element_weights, channel_weights):
    N, C, K = node_feats_1.shape
    G = data_external_field.shape[0]
    E = node_attrs.shape[1]
    feats_t = jnp.transpose(node_feats_1, (2, 0, 1))
    attrs_t = jnp.transpose(node_attrs, (1, 0))
    batch_r = batch.astype(jnp.int32).reshape(1, N)
    fieldx = (data_external_field[:, :, None]
              * channel_weights[None, None, :]).reshape(G, K * C)
    fieldx_bf16 = fieldx.astype(jnp.bfloat16)
    ew_pad = jnp.zeros((1, 128), jnp.float32).at[0, :E].set(element_weights)

    nb = (N + N_BLOCK - 1) // N_BLOCK
    out3 = pl.pallas_call(
        _embed_kernel,
        grid=(nb,),
        in_specs=[
            pl.BlockSpec((1, N_BLOCK), lambda i: (0, i)),
            pl.BlockSpec((E, N_BLOCK), lambda i: (0, i)),
            pl.BlockSpec((K, N_BLOCK, C), lambda i: (0, i, 0)),
            pl.BlockSpec((G, K * C), lambda i: (0, 0)),
            pl.BlockSpec((1, 128), lambda i: (0, 0)),
        ],
        out_specs=pl.BlockSpec((K, N_BLOCK, C), lambda i: (0, i, 0)),
        out_shape=jax.ShapeDtypeStruct((K, N, C), jnp.float32),
    )(batch_r, attrs_t, feats_t, fieldx_bf16, ew_pad)
    return jnp.transpose(out3, (1, 2, 0))


# speedup vs baseline: 11.2078x; 1.0207x over previous
import jax
import jax.numpy as jnp
from jax.experimental import pallas as pl

N_BLOCK = 7936


def _embed_kernel(batch_ref, attrs_ref, feats_ref, fieldx_ref, ew_ref, out_ref):
    B = batch_ref.shape[1]
    G = fieldx_ref.shape[0]
    C = feats_ref.shape[2]
    K = feats_ref.shape[0]
    E = attrs_ref.shape[0]
    a = attrs_ref[...]
    mx = jnp.max(a, axis=0, keepdims=True)
    eq = a == mx
    s_row = jnp.zeros((1, B), jnp.float32)
    taken = jnp.zeros((1, B), jnp.bool_)
    for e in range(E):
        eq_e = eq[e:e + 1, :]
        s_row = jnp.where(eq_e & ~taken, ew_ref[0, e], s_row)
        taken = taken | eq_e
    b_row = batch_ref[...]
    g_ids = jax.lax.broadcasted_iota(jnp.int32, (G, B), 0)
    ohs = jnp.where(g_ids == b_row, s_row, 0.0).astype(jnp.bfloat16)
    mult = jax.lax.dot_general(
        ohs, fieldx_ref[...], (((0,), (0,)), ((), ())),
        preferred_element_type=jnp.float32)
    for k in range(K):
        out_ref[k] = feats_ref[k] + mult[:, k * C:(k + 1) * C]


@jax.jit
def kernel(batch, node_feats_1, node_attrs, data_external_field,
           element_weights, channel_weights):
    N, C, K = node_feats_1.shape
    G = data_external_field.shape[0]
    E = node_attrs.shape[1]
    feats_t = jnp.transpose(node_feats_1, (2, 0, 1))
    attrs_t = jnp.transpose(node_attrs, (1, 0))
    batch_r = batch.astype(jnp.int32).reshape(1, N)
    fieldx = (data_external_field[:, :, None]
              * channel_weights[None, None, :]).reshape(G, K * C)
    fieldx_bf16 = fieldx.astype(jnp.bfloat16)
    ew_pad = jnp.zeros((1, 128), jnp.float32).at[0, :E].set(element_weights)

    nb = (N + N_BLOCK - 1) // N_BLOCK
    out3 = pl.pallas_call(
        _embed_kernel,
        grid=(nb,),
        in_specs=[
            pl.BlockSpec((1, N_BLOCK), lambda i: (0, i)),
            pl.BlockSpec((E, N_BLOCK), lambda i: (0, i)),
            pl.BlockSpec((K, N_BLOCK, C), lambda i: (0, i, 0)),
            pl.BlockSpec((G, K * C), lambda i: (0, 0)),
            pl.BlockSpec((1, 128), lambda i: (0, 0)),
        ],
        out_specs=pl.BlockSpec((K, N_BLOCK, C), lambda i: (0, i, 0)),
        out_shape=jax.ShapeDtypeStruct((K, N, C), jnp.float32),
    )(batch_r, attrs_t, feats_t, fieldx_bf16, ew_pad)
    return jnp.transpose(out3, (1, 2, 0))
